# baseline (speedup 1.0000x reference)
import functools

import jax
import jax.numpy as jnp
from jax.experimental import pallas as pl


def _mlp_body(x_ref, w1t_ref, w2t_ref, w3t_ref, out_ref):
    h1 = jnp.maximum(
        jnp.dot(x_ref[...], w1t_ref[...], preferred_element_type=jnp.float32), 0.0
    )
    h2 = jnp.maximum(
        jnp.dot(h1, w2t_ref[...], preferred_element_type=jnp.float32), 0.0
    )
    out_ref[...] = jnp.dot(h2, w3t_ref[...], preferred_element_type=jnp.float32)


@functools.partial(jax.jit, static_argnames=("tile_b",))
def _mlp(x, W1, W2, W3, tile_b=1024):
    b, d_in = x.shape
    h = W1.shape[0]
    n_out = W3.shape[0]
    w1t = W1.T
    w2t = W2.T
    w3t = W3.T
    grid = (b // tile_b,)
    return pl.pallas_call(
        _mlp_body,
        grid=grid,
        in_specs=[
            pl.BlockSpec((tile_b, d_in), lambda i: (i, 0)),
            pl.BlockSpec((d_in, h), lambda i: (0, 0)),
            pl.BlockSpec((h, h), lambda i: (0, 0)),
            pl.BlockSpec((h, n_out), lambda i: (0, 0)),
        ],
        out_specs=pl.BlockSpec((tile_b, n_out), lambda i: (i, 0)),
        out_shape=jax.ShapeDtypeStruct((b, n_out), jnp.float32),
    )(x, w1t, w2t, w3t)


def kernel(x, W1, W2, W3):
    return _mlp(x, W1, W2, W3)


# speedup vs baseline: 1.0003x; 1.0003x over previous
import functools

import jax
import jax.numpy as jnp
from jax.experimental import pallas as pl


_PREC = jax.lax.Precision.DEFAULT


def _mlp_body(x_ref, w1t_ref, w2t_ref, w3t_ref, out_ref):
    h1 = jnp.maximum(
        jnp.dot(x_ref[...], w1t_ref[...], precision=_PREC,
                preferred_element_type=jnp.float32), 0.0
    )
    h2 = jnp.maximum(
        jnp.dot(h1, w2t_ref[...], precision=_PREC,
                preferred_element_type=jnp.float32), 0.0
    )
    out_ref[...] = jnp.dot(h2, w3t_ref[...], precision=_PREC,
                           preferred_element_type=jnp.float32)


@functools.partial(jax.jit, static_argnames=("tile_b",))
def _mlp(x, W1, W2, W3, tile_b=1024):
    b, d_in = x.shape
    h = W1.shape[0]
    n_out = W3.shape[0]
    w1t = W1.T
    w2t = W2.T
    w3t = W3.T
    grid = (b // tile_b,)
    return pl.pallas_call(
        _mlp_body,
        grid=grid,
        in_specs=[
            pl.BlockSpec((tile_b, d_in), lambda i: (i, 0)),
            pl.BlockSpec((d_in, h), lambda i: (0, 0)),
            pl.BlockSpec((h, h), lambda i: (0, 0)),
            pl.BlockSpec((h, n_out), lambda i: (0, 0)),
        ],
        out_specs=pl.BlockSpec((tile_b, n_out), lambda i: (i, 0)),
        out_shape=jax.ShapeDtypeStruct((b, n_out), jnp.float32),
    )(x, w1t, w2t, w3t)


def kernel(x, W1, W2, W3):
    return _mlp(x, W1, W2, W3)


# speedup vs baseline: 1.0188x; 1.0185x over previous
import functools

import jax
import jax.numpy as jnp
from jax.experimental import pallas as pl


_PREC = jax.lax.Precision.DEFAULT


def _mlp_body(x_ref, w1t_ref, w2t_ref, w3t_ref, out_ref):
    h1 = jnp.maximum(
        jnp.dot(x_ref[...], w1t_ref[...], precision=_PREC,
                preferred_element_type=jnp.float32), 0.0
    )
    h2 = jnp.maximum(
        jnp.dot(h1, w2t_ref[...], precision=_PREC,
                preferred_element_type=jnp.float32), 0.0
    )
    out_ref[...] = jnp.dot(h2, w3t_ref[...], precision=_PREC,
                           preferred_element_type=jnp.float32)


@functools.partial(jax.jit, static_argnames=("tile_b",))
def _mlp(x, W1, W2, W3, tile_b=1024):
    b, d_in = x.shape
    h = W1.shape[0]
    n_out = W3.shape[0]
    w1t = W1.T
    w2t = W2.T
    w3t = W3.T
    grid = (b // tile_b,)
    return pl.pallas_call(
        _mlp_body,
        grid=grid,
        in_specs=[
            pl.BlockSpec((tile_b, d_in), lambda i: (i, 0)),
            pl.BlockSpec((d_in, h), lambda i: (0, 0)),
            pl.BlockSpec((h, h), lambda i: (0, 0)),
            pl.BlockSpec((h, n_out), lambda i: (0, 0)),
        ],
        out_specs=pl.BlockSpec((tile_b, n_out), lambda i: (i, 0)),
        out_shape=jax.ShapeDtypeStruct((b, n_out), jnp.float32),
    )(x, w1t, w2t, w3t)


def kernel(x, W1, W2, W3):
    return _mlp(x, W1, W2, W3, tile_b=2048)
